# Initial kernel scaffold; baseline (speedup 1.0000x reference)
import jax
import jax.numpy as jnp
from jax.experimental import pallas as pl


def kernel(multimodal_feat, query_feat, W_qe1, b_qe1, W_qe2, b_qe2, W_fg, b_fg, W_g1, W_g2):
    raise NotImplementedError("write your pallas kernel here")


# speedup vs baseline: 1.2607x; 1.2607x over previous
import functools

import jax
import jax.numpy as jnp
from jax.experimental import pallas as pl

T = 32768
D = 768
H = 768
E = 64
G4 = 4 * E

TB = 1024


def _router_body(mm_ref, qf_ref, wqe1_ref, bqe1_ref, wqe2_ref, bqe2_ref,
                 wfg_ref, bfg_ref, wg1_ref, wg2_ref,
                 logits_ref, ew_ref, tkw_ref, tki_ref):
    f32 = jnp.float32
    q = jnp.dot(qf_ref[...], wqe1_ref[...], preferred_element_type=f32)
    q = jnp.maximum(q + bqe1_ref[...], 0.0)
    q = jnp.dot(q, wqe2_ref[...], preferred_element_type=f32) + bqe2_ref[...]

    h = jnp.dot(mm_ref[...], wfg_ref[0:H, :], preferred_element_type=f32)
    h = h + jnp.dot(q, wfg_ref[H:2 * H, :], preferred_element_type=f32)
    h = jnp.maximum(h + bfg_ref[...], 0.0)

    g = jnp.tanh(jnp.dot(h, wg1_ref[...], preferred_element_type=f32))
    logits = jnp.dot(g, wg2_ref[...], preferred_element_type=f32)
    logits_ref[...] = logits

    m = jnp.max(logits, axis=-1, keepdims=True)
    ex = jnp.exp(logits - m)
    ew = ex / jnp.sum(ex, axis=-1, keepdims=True)
    ew_ref[...] = ew

    col = jax.lax.broadcasted_iota(jnp.int32, ew.shape, 1)
    w1 = jnp.max(ew, axis=-1, keepdims=True)
    i1 = jnp.min(jnp.where(ew == w1, col, E), axis=-1, keepdims=True)
    masked = jnp.where(col == i1, -1.0, ew)
    w2 = jnp.max(masked, axis=-1, keepdims=True)
    i2 = jnp.min(jnp.where(masked == w2, col, E), axis=-1, keepdims=True)

    denom = w1 + w2 + 1e-6
    tkw_ref[...] = jnp.concatenate([w1, w2], axis=1) / denom
    tki_ref[...] = jnp.concatenate([i1, i2], axis=1)


@functools.partial(jax.jit, static_argnames=("interpret",))
def _router(mm, qf, W_qe1, b_qe1, W_qe2, b_qe2, W_fg, b_fg, W_g1, W_g2,
            interpret=False):
    grid = (T // TB,)
    tok = lambda i: (i, 0)
    rep = lambda i: (0, 0)
    return pl.pallas_call(
        _router_body,
        grid=grid,
        in_specs=[
            pl.BlockSpec((TB, H), tok),
            pl.BlockSpec((TB, D), tok),
            pl.BlockSpec((D, H), rep),
            pl.BlockSpec((1, H), rep),
            pl.BlockSpec((H, H), rep),
            pl.BlockSpec((1, H), rep),
            pl.BlockSpec((2 * H, H), rep),
            pl.BlockSpec((1, H), rep),
            pl.BlockSpec((H, G4), rep),
            pl.BlockSpec((G4, E), rep),
        ],
        out_specs=[
            pl.BlockSpec((TB, E), tok),
            pl.BlockSpec((TB, E), tok),
            pl.BlockSpec((TB, 2), tok),
            pl.BlockSpec((TB, 2), tok),
        ],
        out_shape=[
            jax.ShapeDtypeStruct((T, E), jnp.float32),
            jax.ShapeDtypeStruct((T, E), jnp.float32),
            jax.ShapeDtypeStruct((T, 2), jnp.float32),
            jax.ShapeDtypeStruct((T, 2), jnp.int32),
        ],
        interpret=interpret,
    )(mm, qf, W_qe1, b_qe1, W_qe2, b_qe2, W_fg, b_fg, W_g1, W_g2)


def kernel(multimodal_feat, query_feat, W_qe1, b_qe1, W_qe2, b_qe2,
           W_fg, b_fg, W_g1, W_g2):
    logits, ew, tkw, tki = _router(
        multimodal_feat, query_feat,
        W_qe1, b_qe1.reshape(1, H),
        W_qe2, b_qe2.reshape(1, H),
        W_fg, b_fg.reshape(1, H),
        W_g1, W_g2)
    return (logits, ew, tkw, tki)


# speedup vs baseline: 1.3400x; 1.0629x over previous
import functools

import jax
import jax.numpy as jnp
from jax.experimental import pallas as pl

T = 32768
D = 768
H = 768
E = 64
G4 = 4 * E

TB = 2048


def _router_body(mm_ref, qf_ref, wqe1_ref, bqe1_ref, wqe2_ref, bqe2_ref,
                 wfg_ref, bfg_ref, wg1_ref, wg2_ref,
                 logits_ref, ew_ref, tkw_ref, tki_ref):
    f32 = jnp.float32
    q = jnp.dot(qf_ref[...], wqe1_ref[...], preferred_element_type=f32)
    q = jnp.maximum(q + bqe1_ref[...], 0.0)
    q = jnp.dot(q, wqe2_ref[...], preferred_element_type=f32) + bqe2_ref[...]

    h = jnp.dot(mm_ref[...], wfg_ref[0:H, :], preferred_element_type=f32)
    h = h + jnp.dot(q, wfg_ref[H:2 * H, :], preferred_element_type=f32)
    h = jnp.maximum(h + bfg_ref[...], 0.0)

    g = jnp.tanh(jnp.dot(h, wg1_ref[...], preferred_element_type=f32))
    logits = jnp.dot(g, wg2_ref[...], preferred_element_type=f32)
    logits_ref[...] = logits

    m1 = jnp.max(logits, axis=-1, keepdims=True)
    ex = jnp.exp(logits - m1)
    z = jnp.sum(ex, axis=-1, keepdims=True)
    ew_ref[...] = ex / z

    col = jax.lax.broadcasted_iota(jnp.int32, logits.shape, 1)
    i1 = jnp.min(jnp.where(logits == m1, col, E), axis=-1, keepdims=True)
    masked = jnp.where(col == i1, -jnp.inf, logits)
    m2 = jnp.max(masked, axis=-1, keepdims=True)
    i2 = jnp.min(jnp.where(masked == m2, col, E), axis=-1, keepdims=True)

    t2 = jnp.exp(m2 - m1)
    denom = 1.0 + t2 + 1e-6 * z
    tkw_ref[...] = jnp.concatenate([jnp.ones_like(t2), t2], axis=1) / denom
    tki_ref[...] = jnp.concatenate([i1, i2], axis=1)


@functools.partial(jax.jit, static_argnames=("interpret",))
def _router(mm, qf, W_qe1, b_qe1, W_qe2, b_qe2, W_fg, b_fg, W_g1, W_g2,
            interpret=False):
    grid = (T // TB,)
    tok = lambda i: (i, 0)
    rep = lambda i: (0, 0)
    return pl.pallas_call(
        _router_body,
        grid=grid,
        in_specs=[
            pl.BlockSpec((TB, H), tok),
            pl.BlockSpec((TB, D), tok),
            pl.BlockSpec((D, H), rep),
            pl.BlockSpec((1, H), rep),
            pl.BlockSpec((H, H), rep),
            pl.BlockSpec((1, H), rep),
            pl.BlockSpec((2 * H, H), rep),
            pl.BlockSpec((1, H), rep),
            pl.BlockSpec((H, G4), rep),
            pl.BlockSpec((G4, E), rep),
        ],
        out_specs=[
            pl.BlockSpec((TB, E), tok),
            pl.BlockSpec((TB, E), tok),
            pl.BlockSpec((TB, 2), tok),
            pl.BlockSpec((TB, 2), tok),
        ],
        out_shape=[
            jax.ShapeDtypeStruct((T, E), jnp.float32),
            jax.ShapeDtypeStruct((T, E), jnp.float32),
            jax.ShapeDtypeStruct((T, 2), jnp.float32),
            jax.ShapeDtypeStruct((T, 2), jnp.int32),
        ],
        interpret=interpret,
    )(mm, qf, W_qe1, b_qe1, W_qe2, b_qe2, W_fg, b_fg, W_g1, W_g2)


def kernel(multimodal_feat, query_feat, W_qe1, b_qe1, W_qe2, b_qe2,
           W_fg, b_fg, W_g1, W_g2):
    logits, ew, tkw, tki = _router(
        multimodal_feat, query_feat,
        W_qe1, b_qe1.reshape(1, H),
        W_qe2, b_qe2.reshape(1, H),
        W_fg, b_fg.reshape(1, H),
        W_g1, W_g2)
    return (logits, ew, tkw, tki)
